# baseline (speedup 1.0000x reference)
import functools

import jax
import jax.numpy as jnp
from jax import lax
from jax.experimental import pallas as pl
from jax.experimental.pallas import tpu as pltpu
from jax.experimental.pallas import tpu_sc as plsc

B = 16384
NS, L = 16, 16
BW = B // NS

OFFS = (0, 7, 13, 25, 29, 35, 40, 44)
TBL = 58


def _body(cs, cu, cc, br, ga, gs, gz, gc, d_hbm, p0_hbm, p1_hbm,
          idx_v, d_v, p0_v, p1_v, sem):
    wid = lax.axis_index("s")
    base = wid * BW

    feats = (cs, cu, cc, br, ga, gs, gz, gc)
    dcopy = pltpu.async_copy(d_hbm, d_v, sem)
    copies = [pltpu.async_copy(feats[f].at[pl.ds(base, BW)], idx_v.at[f], sem)
              for f in range(8)]
    dcopy.wait()
    for c in copies:
        c.wait()

    zero = jnp.zeros((L,), jnp.int32)
    bd = plsc.load_gather(d_v, [zero + TBL])

    @plsc.parallel_loop(0, BW, step=L)
    def _chunk(i):
        acc = bd
        for f in range(8):
            x = idx_v[f, pl.ds(i, L)]
            acc = acc + plsc.load_gather(d_v, [x + OFFS[f]])
        p0 = 1.0 / (1.0 + jnp.exp(-acc))
        p0_v[pl.ds(i, L)] = p0
        p1_v[pl.ds(i, L)] = 1.0 - p0

    out0 = pltpu.async_copy(p0_v, p0_hbm.at[pl.ds(base, BW)], sem)
    out1 = pltpu.async_copy(p1_v, p1_hbm.at[pl.ds(base, BW)], sem)
    out0.wait()
    out1.wait()


_mushroom_sc = functools.partial(
    pl.kernel,
    out_type=(jax.ShapeDtypeStruct((B,), jnp.float32),
              jax.ShapeDtypeStruct((B,), jnp.float32)),
    mesh=plsc.VectorSubcoreMesh(core_axis_name="c", subcore_axis_name="s",
                                num_cores=1),
    compiler_params=pltpu.CompilerParams(
        needs_layout_passes=False,
        disable_bounds_checks=True,
        disable_semaphore_checks=True,
        skip_device_barrier=True,
    ),
    scratch_types=[
        pltpu.VMEM((8, BW), jnp.int32),
        pltpu.VMEM((64,), jnp.float32),
        pltpu.VMEM((BW,), jnp.float32),
        pltpu.VMEM((BW,), jnp.float32),
        pltpu.SemaphoreType.DMA,
    ],
)(_body)


def kernel(cap_shape, cap_surface, cap_color, bruises, odor, gill_attachment,
           gill_spacing, gill_size, gill_color, W, b):
    del odor
    idxs = [x.astype(jnp.int32) for x in
            (cap_shape, cap_surface, cap_color, bruises, gill_attachment,
             gill_spacing, gill_size, gill_color)]
    w = W.astype(jnp.float32)
    bd = (b[0] - b[1]).astype(jnp.float32)
    d = jnp.broadcast_to(bd, (64,)).at[0:TBL].set(w[:, 0] - w[:, 1])
    p0, p1 = _mushroom_sc(*idxs, d)
    return jnp.stack([p0, p1], axis=1)


# speedup vs baseline: 1.1001x; 1.1001x over previous
import functools

import jax
import jax.numpy as jnp
from jax import lax
from jax.experimental import pallas as pl
from jax.experimental.pallas import tpu as pltpu
from jax.experimental.pallas import tpu_sc as plsc

B = 16384
NS, L = 16, 16
BW = B // NS

OFFS = (0, 7, 13, 25, 29, 35, 40, 44)
TBL = 58


def _body(cs, cu, cc, br, ga, gs, gz, gc, d_hbm, p0_hbm, p1_hbm,
          idx_v, d_v, p0_v, p1_v, sem):
    wid = lax.axis_index("s")
    base = wid * BW

    feats = (cs, cu, cc, br, ga, gs, gz, gc)
    dcopy = pltpu.async_copy(d_hbm, d_v, sem)
    copies = [pltpu.async_copy(feats[f].at[pl.ds(base, BW)], idx_v.at[f], sem)
              for f in range(8)]
    dcopy.wait()
    for c in copies:
        c.wait()

    zero = jnp.zeros((L,), jnp.int32)
    bd = plsc.load_gather(d_v, [zero + TBL])

    @plsc.parallel_loop(0, BW, step=L)
    def _chunk(i):
        acc = bd
        for f in range(8):
            x = idx_v[f, pl.ds(i, L)]
            acc = acc + plsc.load_gather(d_v, [x + OFFS[f]])
        p0 = 1.0 / (1.0 + jnp.exp(-acc))
        s = (i >> 7) * 256 + (i & 127)
        p_v[pl.ds(s, L)] = p0
        p_v[pl.ds(s + 128, L)] = 1.0 - p0

    pltpu.sync_copy(p_v, p_hbm.at[pl.ds(base * 2, BW * 2)])


_mushroom_sc = functools.partial(
    pl.kernel,
    out_type=jax.ShapeDtypeStruct((B * 2,), jnp.float32),
    mesh=plsc.VectorSubcoreMesh(core_axis_name="c", subcore_axis_name="s",
                                num_cores=1),
    compiler_params=pltpu.CompilerParams(
        needs_layout_passes=False,
        disable_bounds_checks=True,
        disable_semaphore_checks=True,
        skip_device_barrier=True,
    ),
    scratch_types=[
        pltpu.VMEM((8, BW), jnp.int32),
        pltpu.VMEM((64,), jnp.float32),
        pltpu.VMEM((BW * 2,), jnp.float32),
        pltpu.SemaphoreType.DMA,
    ],
)(_body)


def kernel(cap_shape, cap_surface, cap_color, bruises, odor, gill_attachment,
           gill_spacing, gill_size, gill_color, W, b):
    del odor
    idxs = [x.astype(jnp.int32) for x in
            (cap_shape, cap_surface, cap_color, bruises, gill_attachment,
             gill_spacing, gill_size, gill_color)]
    w = W.astype(jnp.float32)
    bd = (b[0] - b[1]).astype(jnp.float32)
    d = jnp.broadcast_to(bd, (64,)).at[0:TBL].set(w[:, 0] - w[:, 1])
    p = _mushroom_sc(*idxs, d)
    return p.reshape(B // 128, 2, 128).transpose(0, 2, 1).reshape(B, 2)
